# baseline (speedup 1.0000x reference)
import functools

import jax
import jax.numpy as jnp
from jax import lax
from jax.experimental import pallas as pl
from jax.experimental.pallas import tpu as pltpu
from jax.experimental.pallas import tpu_sc as plsc

N = 10000
E = 320000
K = 5

_NC = 2
_NS = 16
_NW = _NC * _NS
_EPW = E // _NW
_C = 80
_NCH = _EPW // _C
_NPAD = 10240
_NPS = _NPAD // _NS

_TN = 1000

_W = 320
_NSB = _W // 64


def _prep_body(src_ref, attr_ref, i0_ref, w0_ref):
    a = attr_ref[...]
    v = jnp.clip(a, 0.0, 1.0) * (K - 1)
    k0 = jnp.minimum(jnp.floor(v), float(K - 2))
    frac = v - k0
    k0i = k0.astype(jnp.int32)
    i0_ref[...] = k0i * N + src_ref[...]
    w0_ref[...] = 1.0 - frac


def _prep(src2d, attr2d):
    sh = src2d.shape
    return pl.pallas_call(
        _prep_body,
        out_shape=(
            jax.ShapeDtypeStruct(sh, jnp.int32),
            jax.ShapeDtypeStruct(sh, jnp.float32),
        ),
    )(src2d, attr2d)


def _mm_first_body(x_ref, w_ref, out_ref):
    out_ref[0] = jnp.dot(x_ref[...], w_ref[0],
                         preferred_element_type=jnp.float32)


def _mm_first(x, wstack):
    kk, din, dout = wstack.shape
    grid = (kk, N // _TN)
    return pl.pallas_call(
        _mm_first_body,
        grid=grid,
        in_specs=[
            pl.BlockSpec((_TN, din), lambda k, n: (n, 0)),
            pl.BlockSpec((1, din, dout), lambda k, n: (k, 0, 0)),
        ],
        out_specs=pl.BlockSpec((1, _TN, dout), lambda k, n: (k, n, 0)),
        out_shape=jax.ShapeDtypeStruct((kk, N, dout), jnp.float32),
    )(x, wstack)


def _elu(v):
    return jnp.where(v > 0, v, jnp.exp(v) - 1.0)


def _mm_fused_body(agg_ref, r_ref, b_ref, w_ref, out_ref):
    h = _elu(agg_ref[0] + agg_ref[1] + r_ref[...] + b_ref[...])
    out_ref[0] = jnp.dot(h, w_ref[0], preferred_element_type=jnp.float32)


def _mm_fused(agg, r, bias, wstack):
    kk, din, dout = wstack.shape
    grid = (kk, N // _TN)
    return pl.pallas_call(
        _mm_fused_body,
        grid=grid,
        in_specs=[
            pl.BlockSpec((2, _TN, din), lambda k, n: (0, n, 0)),
            pl.BlockSpec((_TN, din), lambda k, n: (n, 0)),
            pl.BlockSpec((1, din), lambda k, n: (0, 0)),
            pl.BlockSpec((1, din, dout), lambda k, n: (k, 0, 0)),
        ],
        out_specs=pl.BlockSpec((1, _TN, dout), lambda k, n: (k, n, 0)),
        out_shape=jax.ShapeDtypeStruct((kk, N, dout), jnp.float32),
    )(agg, r, bias.reshape(1, din), wstack)


def _head_body(agg_ref, r_ref, b_ref, w1_ref, b1_ref, w2_ref, b2_ref, out_ref):
    h = _elu(agg_ref[0] + agg_ref[1] + r_ref[...] + b_ref[...])
    t = _elu(jnp.dot(h, w1_ref[...], preferred_element_type=jnp.float32)
             + b1_ref[...])
    logits = (jnp.dot(t, w2_ref[...], preferred_element_type=jnp.float32)
              + b2_ref[...])
    m = jnp.max(logits, axis=1, keepdims=True)
    lse = jnp.log(jnp.sum(jnp.exp(logits - m), axis=1, keepdims=True)) + m
    out_ref[...] = logits - lse


def _head(agg, r, bias, lin1_W, lin1_b, lin2_W, lin2_b):
    din = r.shape[1]
    d1 = lin1_W.shape[1]
    d2 = lin2_W.shape[1]
    grid = (N // _TN,)
    return pl.pallas_call(
        _head_body,
        grid=grid,
        in_specs=[
            pl.BlockSpec((2, _TN, din), lambda n: (0, n, 0)),
            pl.BlockSpec((_TN, din), lambda n: (n, 0)),
            pl.BlockSpec((1, din), lambda n: (0, 0)),
            pl.BlockSpec((din, d1), lambda n: (0, 0)),
            pl.BlockSpec((1, d1), lambda n: (0, 0)),
            pl.BlockSpec((d1, d2), lambda n: (0, 0)),
            pl.BlockSpec((1, d2), lambda n: (0, 0)),
        ],
        out_specs=pl.BlockSpec((_TN, d2), lambda n: (n, 0)),
        out_shape=jax.ShapeDtypeStruct((N, d2), jnp.float32),
    )(agg, r, bias.reshape(1, din), lin1_W, lin1_b.reshape(1, d1),
      lin2_W, lin2_b.reshape(1, d2))


def _sc_agg_body(dout, y_hbm, i0_hbm, dst_hbm, w0_hbm,
                 out_hbm, i0_v, i1_v, dst_v, w0_v, rows0, rows1,
                 wloc, fidx, agg_sp, sem0, sem1):
    cid = lax.axis_index("c")
    sid = lax.axis_index("s")
    wid = cid * _NS + sid

    pltpu.sync_copy(i0_hbm.at[wid], i0_v)
    pltpu.sync_copy(dst_hbm.at[wid], dst_v)
    pltpu.sync_copy(w0_hbm.at[wid], w0_v)

    nq = dout // 16
    lane = lax.iota(jnp.int32, 16)
    zv = jnp.zeros((16,), jnp.float32)

    def zero_wloc():
        def zw(r, c):
            for q in range(nq):
                wloc[r, pl.ds(q * 16, 16)] = zv
            return c

        lax.fori_loop(0, _W, zw, 0)

    zero_wloc()

    def zr(r, c):
        for q in range(nq):
            rows0[0, r, pl.ds(q * 16, 16)] = zv
        return c

    lax.fori_loop(0, _C, zr, 0)
    for t in range(_NPS // _C):
        pltpu.sync_copy(rows0.at[0],
                        agg_sp.at[pl.ds(sid * _NPS + t * _C, _C)])

    def ini(r, c):
        for p in range(_C // 16):
            sl = pl.ds(p * 16, 16)
            i1_v[r, sl] = i0_v[r, sl] + N
        return c

    lax.fori_loop(0, _NCH, ini, 0)
    plsc.subcore_barrier()

    def flush(base):
        for j in range(_NSB):
            for q4 in range(4):
                fidx[j, pl.ds(q4 * 16, 16)] = base + (j * 64 + q4 * 16) + lane
        for j in range(_NSB):
            pltpu.sync_copy(wloc.at[pl.ds(j * 64, 64)], agg_sp.at[fidx.at[j]],
                            add=True)
        zero_wloc()

    def issue(ch, b):
        pltpu.async_copy(y_hbm.at[i0_v.at[ch]], rows0.at[b], sem0)
        pltpu.async_copy(y_hbm.at[i1_v.at[ch]], rows1.at[b], sem1)

    issue(0, 0)
    base_init = dst_v[0, pl.ds(0, 16)][0]

    def chunk(ch, base):
        b = lax.rem(ch, 2)
        pltpu.make_async_copy(y_hbm.at[i0_v.at[ch]], rows0.at[b], sem0).wait()
        pltpu.make_async_copy(y_hbm.at[i1_v.at[ch]], rows1.at[b], sem1).wait()

        @pl.when(ch + 1 < _NCH)
        def _():
            issue(ch + 1, 1 - b)

        d_first = dst_v[ch, pl.ds(0, 16)][0]
        d_last = dst_v[ch, pl.ds(_C - 16, 16)][15]
        wide = (d_last - d_first) > (_W - 1)
        need = jnp.logical_and(jnp.logical_not(wide), d_last >= base + _W)

        @pl.when(need)
        def _():
            flush(base)

        base2 = jnp.where(need, d_first, base)

        @pl.when(wide)
        def _():
            def grp_direct(g, c2):
                wv0 = w0_v[ch, pl.ds(g * 16, 16)]
                wv1 = 1.0 - wv0
                for j in range(16):
                    rr = g * 16 + j
                    a = wv0[j]
                    bb = wv1[j]
                    for q in range(nq):
                        sl = pl.ds(q * 16, 16)
                        rows0[b, rr, sl] = (a * rows0[b, rr, sl]
                                            + bb * rows1[b, rr, sl])
                return c2

            lax.fori_loop(0, _C // 16, grp_direct, 0)
            pltpu.sync_copy(rows0.at[b], agg_sp.at[dst_v.at[ch]], add=True)

        @pl.when(jnp.logical_not(wide))
        def _():
            def grp_win(g, c2):
                wv0 = w0_v[ch, pl.ds(g * 16, 16)]
                wv1 = 1.0 - wv0
                dl = dst_v[ch, pl.ds(g * 16, 16)] - base2
                for j in range(16):
                    rr = g * 16 + j
                    a = wv0[j]
                    bb = wv1[j]
                    dd = dl[j]
                    for q in range(nq):
                        sl = pl.ds(q * 16, 16)
                        wloc[dd, sl] = (wloc[dd, sl]
                                        + a * rows0[b, rr, sl]
                                        + bb * rows1[b, rr, sl])
                return c2

            lax.fori_loop(0, _C // 16, grp_win, 0)

        return base2

    base_fin = lax.fori_loop(0, _NCH, chunk, base_init)
    flush(base_fin)
    plsc.subcore_barrier()

    pltpu.sync_copy(agg_sp.at[pl.ds(sid * _NPS, _NPS)],
                    out_hbm.at[cid, pl.ds(sid * _NPS, _NPS)])


@functools.partial(jax.jit, static_argnames=("dout",))
def _sc_agg(y, i0, dst, w0, *, dout):
    mesh = plsc.VectorSubcoreMesh(core_axis_name="c", subcore_axis_name="s")
    body = functools.partial(_sc_agg_body, dout)
    return pl.kernel(
        body,
        out_type=pltpu.HBM((_NC, _NPAD, dout), jnp.float32),
        mesh=mesh,
        scratch_types=[
            pltpu.VMEM((_NCH, _C), jnp.int32),
            pltpu.VMEM((_NCH, _C), jnp.int32),
            pltpu.VMEM((_NCH, _C), jnp.int32),
            pltpu.VMEM((_NCH, _C), jnp.float32),
            pltpu.VMEM((2, _C, dout), jnp.float32),
            pltpu.VMEM((2, _C, dout), jnp.float32),
            pltpu.VMEM((_W, dout), jnp.float32),
            pltpu.VMEM((_NSB, 64), jnp.int32),
            pltpu.VMEM_SHARED((_NPAD, dout), jnp.float32),
            pltpu.SemaphoreType.DMA,
            pltpu.SemaphoreType.DMA,
        ],
        compiler_params=pltpu.CompilerParams(use_tc_tiling_on_sc=False),
    )(y, i0, dst, w0)


def kernel(x, edge_index, edge_attr,
           conv1_W, conv1_root, conv1_bias,
           conv2_W, conv2_root, conv2_bias,
           conv3_W, conv3_root, conv3_bias,
           conv4_W, conv4_root, conv4_bias,
           conv5_W, conv5_root, conv5_bias,
           conv6_W, conv6_root, conv6_bias,
           lin1_W, lin1_b, lin2_W, lin2_b):
    src = edge_index[0]
    dst = edge_index[1]
    attr = edge_attr[:, 0]

    i0, w0 = _prep(src.reshape(2500, 128), attr.reshape(2500, 128))
    dst, i0, w0 = lax.sort(
        (dst, i0.reshape(-1), w0.reshape(-1)), num_keys=1)
    i0 = i0.reshape(_NW, _NCH, _C)
    w0 = w0.reshape(_NW, _NCH, _C)
    dst2 = dst.reshape(_NW, _NCH, _C)

    convs = [
        (conv1_W, conv1_root, conv1_bias),
        (conv2_W, conv2_root, conv2_bias),
        (conv3_W, conv3_root, conv3_bias),
        (conv4_W, conv4_root, conv4_bias),
        (conv5_W, conv5_root, conv5_bias),
        (conv6_W, conv6_root, conv6_bias),
    ]

    wstack = jnp.concatenate([convs[0][0], convs[0][1][None]], axis=0)
    y = _mm_first(x, wstack)
    dout = wstack.shape[2]
    agg = _sc_agg(y.reshape(-1, dout), i0, dst2, w0, dout=dout)
    r = y[K]
    prev_bias = convs[0][2]

    for W, root, bias in convs[1:]:
        wstack = jnp.concatenate([W, root[None]], axis=0)
        y = _mm_fused(agg, r, prev_bias, wstack)
        dout = wstack.shape[2]
        agg = _sc_agg(y.reshape(-1, dout), i0, dst2, w0, dout=dout)
        r = y[K]
        prev_bias = bias

    return _head(agg, r, prev_bias, lin1_W, lin1_b, lin2_W, lin2_b)


# speedup vs baseline: 1.2274x; 1.2274x over previous
import functools

import jax
import jax.numpy as jnp
from jax import lax
from jax.experimental import pallas as pl
from jax.experimental.pallas import tpu as pltpu
from jax.experimental.pallas import tpu_sc as plsc

N = 10000
E = 320000
K = 5

_NC = 2
_NS = 16
_NW = _NC * _NS
_EPW = E // _NW
_C = 80
_NCH = _EPW // _C
_NPAD = 10240
_NPS = _NPAD // _NS

_TN = 1000


def _prep_body(src_ref, attr_ref, i0_ref, w0_ref):
    a = attr_ref[...]
    v = jnp.clip(a, 0.0, 1.0) * (K - 1)
    k0 = jnp.minimum(jnp.floor(v), float(K - 2))
    frac = v - k0
    k0i = k0.astype(jnp.int32)
    i0_ref[...] = k0i * N + src_ref[...]
    w0_ref[...] = 1.0 - frac


def _prep(src2d, attr2d):
    sh = src2d.shape
    return pl.pallas_call(
        _prep_body,
        out_shape=(
            jax.ShapeDtypeStruct(sh, jnp.int32),
            jax.ShapeDtypeStruct(sh, jnp.float32),
        ),
    )(src2d, attr2d)


def _mm_first_body(x_ref, w_ref, out_ref):
    out_ref[0] = jnp.dot(x_ref[...], w_ref[0],
                         preferred_element_type=jnp.float32)


def _mm_first(x, wstack):
    kk, din, dout = wstack.shape
    grid = (kk, N // _TN)
    return pl.pallas_call(
        _mm_first_body,
        grid=grid,
        in_specs=[
            pl.BlockSpec((_TN, din), lambda k, n: (n, 0)),
            pl.BlockSpec((1, din, dout), lambda k, n: (k, 0, 0)),
        ],
        out_specs=pl.BlockSpec((1, _TN, dout), lambda k, n: (k, n, 0)),
        out_shape=jax.ShapeDtypeStruct((kk, N, dout), jnp.float32),
    )(x, wstack)


def _elu(v):
    return jnp.where(v > 0, v, jnp.exp(v) - 1.0)


def _mm_fused_body(agg_ref, r_ref, b_ref, w_ref, out_ref):
    h = _elu(agg_ref[0] + agg_ref[1] + r_ref[...] + b_ref[...])
    out_ref[0] = jnp.dot(h, w_ref[0], preferred_element_type=jnp.float32)


def _mm_fused(agg, r, bias, wstack):
    kk, din, dout = wstack.shape
    grid = (kk, N // _TN)
    return pl.pallas_call(
        _mm_fused_body,
        grid=grid,
        in_specs=[
            pl.BlockSpec((2, _TN, din), lambda k, n: (0, n, 0)),
            pl.BlockSpec((_TN, din), lambda k, n: (n, 0)),
            pl.BlockSpec((1, din), lambda k, n: (0, 0)),
            pl.BlockSpec((1, din, dout), lambda k, n: (k, 0, 0)),
        ],
        out_specs=pl.BlockSpec((1, _TN, dout), lambda k, n: (k, n, 0)),
        out_shape=jax.ShapeDtypeStruct((kk, N, dout), jnp.float32),
    )(agg, r, bias.reshape(1, din), wstack)


def _head_body(agg_ref, r_ref, b_ref, w1_ref, b1_ref, w2_ref, b2_ref, out_ref):
    h = _elu(agg_ref[0] + agg_ref[1] + r_ref[...] + b_ref[...])
    t = _elu(jnp.dot(h, w1_ref[...], preferred_element_type=jnp.float32)
             + b1_ref[...])
    logits = (jnp.dot(t, w2_ref[...], preferred_element_type=jnp.float32)
              + b2_ref[...])
    m = jnp.max(logits, axis=1, keepdims=True)
    lse = jnp.log(jnp.sum(jnp.exp(logits - m), axis=1, keepdims=True)) + m
    out_ref[...] = logits - lse


def _head(agg, r, bias, lin1_W, lin1_b, lin2_W, lin2_b):
    din = r.shape[1]
    d1 = lin1_W.shape[1]
    d2 = lin2_W.shape[1]
    grid = (N // _TN,)
    return pl.pallas_call(
        _head_body,
        grid=grid,
        in_specs=[
            pl.BlockSpec((2, _TN, din), lambda n: (0, n, 0)),
            pl.BlockSpec((_TN, din), lambda n: (n, 0)),
            pl.BlockSpec((1, din), lambda n: (0, 0)),
            pl.BlockSpec((din, d1), lambda n: (0, 0)),
            pl.BlockSpec((1, d1), lambda n: (0, 0)),
            pl.BlockSpec((d1, d2), lambda n: (0, 0)),
            pl.BlockSpec((1, d2), lambda n: (0, 0)),
        ],
        out_specs=pl.BlockSpec((_TN, d2), lambda n: (n, 0)),
        out_shape=jax.ShapeDtypeStruct((N, d2), jnp.float32),
    )(agg, r, bias.reshape(1, din), lin1_W, lin1_b.reshape(1, d1),
      lin2_W, lin2_b.reshape(1, d2))


def _sc_agg_body(dout, y_hbm, i0_hbm, dst_hbm, w0_hbm,
                 out_hbm, i0_v, i1_v, dst_v, w0_v, rows0, rows1,
                 agg_sp, sem0, sem1, sem2):
    cid = lax.axis_index("c")
    sid = lax.axis_index("s")
    wid = cid * _NS + sid

    pltpu.sync_copy(i0_hbm.at[wid], i0_v)
    pltpu.sync_copy(dst_hbm.at[wid], dst_v)
    pltpu.sync_copy(w0_hbm.at[wid], w0_v)

    nq = dout // 16
    zv = jnp.zeros((16,), jnp.float32)

    def zr(r, c):
        for q in range(nq):
            rows0[0, r, pl.ds(q * 16, 16)] = zv
        return c

    lax.fori_loop(0, _C, zr, 0)
    for t in range(_NPS // _C):
        pltpu.sync_copy(rows0.at[0],
                        agg_sp.at[pl.ds(sid * _NPS + t * _C, _C)])

    def ini(r, c):
        for p in range(_C // 16):
            sl = pl.ds(p * 16, 16)
            i1_v[r, sl] = i0_v[r, sl] + N
        return c

    lax.fori_loop(0, _NCH, ini, 0)
    plsc.subcore_barrier()

    def issue(ch, b):
        pltpu.async_copy(y_hbm.at[i0_v.at[ch]], rows0.at[b], sem0)
        pltpu.async_copy(y_hbm.at[i1_v.at[ch]], rows1.at[b], sem1)

    issue(0, 0)

    def chunk(ch, carry):
        b = lax.rem(ch, 2)
        pltpu.make_async_copy(y_hbm.at[i0_v.at[ch]], rows0.at[b], sem0).wait()
        pltpu.make_async_copy(y_hbm.at[i1_v.at[ch]], rows1.at[b], sem1).wait()

        @pl.when(ch >= 1)
        def _():
            pltpu.make_async_copy(rows0.at[1 - b],
                                  agg_sp.at[dst_v.at[ch - 1]], sem2).wait()

        @pl.when(ch + 1 < _NCH)
        def _():
            issue(ch + 1, 1 - b)

        def grp(g, c2):
            wv0 = w0_v[ch, pl.ds(g * 16, 16)]
            wv1 = 1.0 - wv0
            for j in range(16):
                rr = g * 16 + j
                a = wv0[j]
                bb = wv1[j]
                for q in range(nq):
                    sl = pl.ds(q * 16, 16)
                    rows0[b, rr, sl] = (a * rows0[b, rr, sl]
                                        + bb * rows1[b, rr, sl])
            return c2

        lax.fori_loop(0, _C // 16, grp, 0)
        pltpu.async_copy(rows0.at[b], agg_sp.at[dst_v.at[ch]], sem2,
                         add=True)
        return carry

    lax.fori_loop(0, _NCH, chunk, 0)
    bl = (_NCH - 1) % 2
    pltpu.make_async_copy(rows0.at[bl], agg_sp.at[dst_v.at[_NCH - 1]],
                          sem2).wait()
    plsc.subcore_barrier()

    pltpu.sync_copy(agg_sp.at[pl.ds(sid * _NPS, _NPS)],
                    out_hbm.at[cid, pl.ds(sid * _NPS, _NPS)])


@functools.partial(jax.jit, static_argnames=("dout",))
def _sc_agg(y, i0, dst, w0, *, dout):
    mesh = plsc.VectorSubcoreMesh(core_axis_name="c", subcore_axis_name="s")
    body = functools.partial(_sc_agg_body, dout)
    return pl.kernel(
        body,
        out_type=pltpu.HBM((_NC, _NPAD, dout), jnp.float32),
        mesh=mesh,
        scratch_types=[
            pltpu.VMEM((_NCH, _C), jnp.int32),
            pltpu.VMEM((_NCH, _C), jnp.int32),
            pltpu.VMEM((_NCH, _C), jnp.int32),
            pltpu.VMEM((_NCH, _C), jnp.float32),
            pltpu.VMEM((2, _C, dout), jnp.float32),
            pltpu.VMEM((2, _C, dout), jnp.float32),
            pltpu.VMEM_SHARED((_NPAD, dout), jnp.float32),
            pltpu.SemaphoreType.DMA,
            pltpu.SemaphoreType.DMA,
            pltpu.SemaphoreType.DMA,
        ],
        compiler_params=pltpu.CompilerParams(use_tc_tiling_on_sc=False),
    )(y, i0, dst, w0)


def kernel(x, edge_index, edge_attr,
           conv1_W, conv1_root, conv1_bias,
           conv2_W, conv2_root, conv2_bias,
           conv3_W, conv3_root, conv3_bias,
           conv4_W, conv4_root, conv4_bias,
           conv5_W, conv5_root, conv5_bias,
           conv6_W, conv6_root, conv6_bias,
           lin1_W, lin1_b, lin2_W, lin2_b):
    src = edge_index[0]
    dst = edge_index[1]
    attr = edge_attr[:, 0]

    i0, w0 = _prep(src.reshape(2500, 128), attr.reshape(2500, 128))
    i0 = i0.reshape(_NW, _NCH, _C)
    w0 = w0.reshape(_NW, _NCH, _C)
    dst2 = dst.reshape(_NW, _NCH, _C)

    convs = [
        (conv1_W, conv1_root, conv1_bias),
        (conv2_W, conv2_root, conv2_bias),
        (conv3_W, conv3_root, conv3_bias),
        (conv4_W, conv4_root, conv4_bias),
        (conv5_W, conv5_root, conv5_bias),
        (conv6_W, conv6_root, conv6_bias),
    ]

    wstack = jnp.concatenate([convs[0][0], convs[0][1][None]], axis=0)
    y = _mm_first(x, wstack)
    dout = wstack.shape[2]
    agg = _sc_agg(y.reshape(-1, dout), i0, dst2, w0, dout=dout)
    r = y[K]
    prev_bias = convs[0][2]

    for W, root, bias in convs[1:]:
        wstack = jnp.concatenate([W, root[None]], axis=0)
        y = _mm_fused(agg, r, prev_bias, wstack)
        dout = wstack.shape[2]
        agg = _sc_agg(y.reshape(-1, dout), i0, dst2, w0, dout=dout)
        r = y[K]
        prev_bias = bias

    return _head(agg, r, prev_bias, lin1_W, lin1_b, lin2_W, lin2_b)


# speedup vs baseline: 2.4048x; 1.9593x over previous
import functools

import jax
import jax.numpy as jnp
from jax import lax
from jax.experimental import pallas as pl
from jax.experimental.pallas import tpu as pltpu
from jax.experimental.pallas import tpu_sc as plsc

N = 10000
E = 320000
K = 5

_NC = 2
_NS = 16
_NW = _NC * _NS
_EPW = E // _NW
_C = 80
_NCH = _EPW // _C
_NPAD = 10240
_NPS = _NPAD // _NS

_TN = 1000


def _prep_body(src_ref, attr_ref, i0_ref, w0_ref):
    a = attr_ref[...]
    v = jnp.clip(a, 0.0, 1.0) * (K - 1)
    k0 = jnp.minimum(jnp.floor(v), float(K - 2))
    frac = v - k0
    k0i = k0.astype(jnp.int32)
    i0_ref[...] = k0i * N + src_ref[...]
    w0_ref[...] = 1.0 - frac


def _prep(src2d, attr2d):
    sh = src2d.shape
    return pl.pallas_call(
        _prep_body,
        out_shape=(
            jax.ShapeDtypeStruct(sh, jnp.int32),
            jax.ShapeDtypeStruct(sh, jnp.float32),
        ),
    )(src2d, attr2d)


def _mm_first_body(x_ref, w_ref, out_ref):
    out_ref[0] = jnp.dot(x_ref[...], w_ref[0],
                         preferred_element_type=jnp.float32)


def _mm_first(x, wstack):
    kk, din, dout = wstack.shape
    grid = (kk, N // _TN)
    return pl.pallas_call(
        _mm_first_body,
        grid=grid,
        in_specs=[
            pl.BlockSpec((_TN, din), lambda k, n: (n, 0)),
            pl.BlockSpec((1, din, dout), lambda k, n: (k, 0, 0)),
        ],
        out_specs=pl.BlockSpec((1, _TN, dout), lambda k, n: (k, n, 0)),
        out_shape=jax.ShapeDtypeStruct((kk, N, dout), jnp.float32),
    )(x, wstack)


def _elu(v):
    return jnp.where(v > 0, v, jnp.exp(v) - 1.0)


def _mm_fused_body(agg_ref, r_ref, b_ref, w_ref, out_ref):
    h = _elu(agg_ref[0] + agg_ref[1] + r_ref[...] + b_ref[...])
    out_ref[0] = jnp.dot(h, w_ref[0], preferred_element_type=jnp.float32)


def _mm_fused(agg, r, bias, wstack):
    kk, din, dout = wstack.shape
    grid = (kk, N // _TN)
    return pl.pallas_call(
        _mm_fused_body,
        grid=grid,
        in_specs=[
            pl.BlockSpec((2, _TN, din), lambda k, n: (0, n, 0)),
            pl.BlockSpec((_TN, din), lambda k, n: (n, 0)),
            pl.BlockSpec((1, din), lambda k, n: (0, 0)),
            pl.BlockSpec((1, din, dout), lambda k, n: (k, 0, 0)),
        ],
        out_specs=pl.BlockSpec((1, _TN, dout), lambda k, n: (k, n, 0)),
        out_shape=jax.ShapeDtypeStruct((kk, N, dout), jnp.float32),
    )(agg, r, bias.reshape(1, din), wstack)


def _head_body(agg_ref, r_ref, b_ref, w1_ref, b1_ref, w2_ref, b2_ref, out_ref):
    h = _elu(agg_ref[0] + agg_ref[1] + r_ref[...] + b_ref[...])
    t = _elu(jnp.dot(h, w1_ref[...], preferred_element_type=jnp.float32)
             + b1_ref[...])
    logits = (jnp.dot(t, w2_ref[...], preferred_element_type=jnp.float32)
              + b2_ref[...])
    m = jnp.max(logits, axis=1, keepdims=True)
    lse = jnp.log(jnp.sum(jnp.exp(logits - m), axis=1, keepdims=True)) + m
    out_ref[...] = logits - lse


def _head(agg, r, bias, lin1_W, lin1_b, lin2_W, lin2_b):
    din = r.shape[1]
    d1 = lin1_W.shape[1]
    d2 = lin2_W.shape[1]
    grid = (N // _TN,)
    return pl.pallas_call(
        _head_body,
        grid=grid,
        in_specs=[
            pl.BlockSpec((2, _TN, din), lambda n: (0, n, 0)),
            pl.BlockSpec((_TN, din), lambda n: (n, 0)),
            pl.BlockSpec((1, din), lambda n: (0, 0)),
            pl.BlockSpec((din, d1), lambda n: (0, 0)),
            pl.BlockSpec((1, d1), lambda n: (0, 0)),
            pl.BlockSpec((d1, d2), lambda n: (0, 0)),
            pl.BlockSpec((1, d2), lambda n: (0, 0)),
        ],
        out_specs=pl.BlockSpec((_TN, d2), lambda n: (n, 0)),
        out_shape=jax.ShapeDtypeStruct((N, d2), jnp.float32),
    )(agg, r, bias.reshape(1, din), lin1_W, lin1_b.reshape(1, d1),
      lin2_W, lin2_b.reshape(1, d2))


_NBUF = 4


def _sc_agg_body(dout, y_hbm, i0_hbm, dst_hbm, w0_hbm,
                 out_hbm, i0_v, i1_v, dst_v, w0_v, rows0, rows1,
                 agg_sp, *sems):
    gsem = sems[:_NBUF]
    ssem = sems[_NBUF:]
    cid = lax.axis_index("c")
    sid = lax.axis_index("s")
    wid = cid * _NS + sid

    pltpu.sync_copy(i0_hbm.at[wid], i0_v)
    pltpu.sync_copy(dst_hbm.at[wid], dst_v)
    pltpu.sync_copy(w0_hbm.at[wid], w0_v)

    nq = dout // 16
    zv = jnp.zeros((16,), jnp.float32)

    def zr(r, c):
        for q in range(nq):
            rows0[0, r, pl.ds(q * 16, 16)] = zv
        return c

    lax.fori_loop(0, _C, zr, 0)
    for t in range(_NPS // _C):
        pltpu.sync_copy(rows0.at[0],
                        agg_sp.at[pl.ds(sid * _NPS + t * _C, _C)])

    def ini(r, c):
        for p in range(_C // 16):
            sl = pl.ds(p * 16, 16)
            i1_v[r, sl] = i0_v[r, sl] + N
        return c

    lax.fori_loop(0, _NCH, ini, 0)
    plsc.subcore_barrier()

    def issue(ch, b):
        pltpu.async_copy(y_hbm.at[i0_v.at[ch]], rows0.at[b], gsem[b])
        pltpu.async_copy(y_hbm.at[i1_v.at[ch]], rows1.at[b], gsem[b])

    for b in range(_NBUF - 1):
        issue(b, b)

    def chunk(b, g5):
        ch = g5 * _NBUF + b
        pltpu.make_async_copy(y_hbm.at[i0_v.at[ch]], rows0.at[b],
                              gsem[b]).wait()
        pltpu.make_async_copy(y_hbm.at[i1_v.at[ch]], rows1.at[b],
                              gsem[b]).wait()

        bb = (b + _NBUF - 1) % _NBUF

        @pl.when(ch + _NBUF - 1 < _NCH)
        def _():
            @pl.when(ch >= 1)
            def _():
                pltpu.make_async_copy(rows0.at[bb],
                                      agg_sp.at[dst_v.at[ch - 1]],
                                      ssem[bb]).wait()

            issue(ch + _NBUF - 1, bb)

        def grp(g, c2):
            wv0 = w0_v[ch, pl.ds(g * 16, 16)]
            wv1 = 1.0 - wv0
            for j in range(16):
                rr = g * 16 + j
                a = wv0[j]
                bbw = wv1[j]
                for q in range(nq):
                    sl = pl.ds(q * 16, 16)
                    rows0[b, rr, sl] = (a * rows0[b, rr, sl]
                                        + bbw * rows1[b, rr, sl])
            return c2

        lax.fori_loop(0, _C // 16, grp, 0)
        pltpu.async_copy(rows0.at[b], agg_sp.at[dst_v.at[ch]], ssem[b],
                         add=True)

    def outer(g5, carry):
        for b in range(_NBUF):
            chunk(b, g5)
        return carry

    lax.fori_loop(0, _NCH // _NBUF, outer, 0)
    for t in range(_NCH - (_NCH // _NBUF) * _NBUF):
        chunk(t, jnp.int32(_NCH // _NBUF))
    for b in range(_NBUF):
        pltpu.make_async_copy(rows0.at[b], agg_sp.at[dst_v.at[0]],
                              ssem[b]).wait()
    plsc.subcore_barrier()

    pltpu.sync_copy(agg_sp.at[pl.ds(sid * _NPS, _NPS)],
                    out_hbm.at[cid, pl.ds(sid * _NPS, _NPS)])


@functools.partial(jax.jit, static_argnames=("dout",))
def _sc_agg(y, i0, dst, w0, *, dout):
    mesh = plsc.VectorSubcoreMesh(core_axis_name="c", subcore_axis_name="s")
    body = functools.partial(_sc_agg_body, dout)
    return pl.kernel(
        body,
        out_type=pltpu.HBM((_NC, _NPAD, dout), jnp.float32),
        mesh=mesh,
        scratch_types=[
            pltpu.VMEM((_NCH, _C), jnp.int32),
            pltpu.VMEM((_NCH, _C), jnp.int32),
            pltpu.VMEM((_NCH, _C), jnp.int32),
            pltpu.VMEM((_NCH, _C), jnp.float32),
            pltpu.VMEM((_NBUF, _C, dout), jnp.float32),
            pltpu.VMEM((_NBUF, _C, dout), jnp.float32),
            pltpu.VMEM_SHARED((_NPAD, dout), jnp.float32),
        ] + [pltpu.SemaphoreType.DMA] * (2 * _NBUF),
        compiler_params=pltpu.CompilerParams(use_tc_tiling_on_sc=False),
    )(y, i0, dst, w0)


def kernel(x, edge_index, edge_attr,
           conv1_W, conv1_root, conv1_bias,
           conv2_W, conv2_root, conv2_bias,
           conv3_W, conv3_root, conv3_bias,
           conv4_W, conv4_root, conv4_bias,
           conv5_W, conv5_root, conv5_bias,
           conv6_W, conv6_root, conv6_bias,
           lin1_W, lin1_b, lin2_W, lin2_b):
    src = edge_index[0]
    dst = edge_index[1]
    attr = edge_attr[:, 0]

    i0, w0 = _prep(src.reshape(2500, 128), attr.reshape(2500, 128))
    i0 = i0.reshape(_NW, _NCH, _C)
    w0 = w0.reshape(_NW, _NCH, _C)
    dst2 = dst.reshape(_NW, _NCH, _C)

    convs = [
        (conv1_W, conv1_root, conv1_bias),
        (conv2_W, conv2_root, conv2_bias),
        (conv3_W, conv3_root, conv3_bias),
        (conv4_W, conv4_root, conv4_bias),
        (conv5_W, conv5_root, conv5_bias),
        (conv6_W, conv6_root, conv6_bias),
    ]

    wstack = jnp.concatenate([convs[0][0], convs[0][1][None]], axis=0)
    y = _mm_first(x, wstack)
    dout = wstack.shape[2]
    agg = _sc_agg(y.reshape(-1, dout), i0, dst2, w0, dout=dout)
    r = y[K]
    prev_bias = convs[0][2]

    for W, root, bias in convs[1:]:
        wstack = jnp.concatenate([W, root[None]], axis=0)
        y = _mm_fused(agg, r, prev_bias, wstack)
        dout = wstack.shape[2]
        agg = _sc_agg(y.reshape(-1, dout), i0, dst2, w0, dout=dout)
        r = y[K]
        prev_bias = bias

    return _head(agg, r, prev_bias, lin1_W, lin1_b, lin2_W, lin2_b)


# speedup vs baseline: 2.5800x; 1.0729x over previous
import functools

import jax
import jax.numpy as jnp
from jax import lax
from jax.experimental import pallas as pl
from jax.experimental.pallas import tpu as pltpu
from jax.experimental.pallas import tpu_sc as plsc

N = 10000
E = 320000
K = 5

_NC = 2
_NS = 16
_NW = _NC * _NS
_EPW = E // _NW
_C = 80
_NCH = _EPW // _C
_NPAD = 10240
_NPS = _NPAD // _NS

_TN = 1000


def _prep_body(src_ref, attr_ref, i0_ref, w0_ref):
    a = attr_ref[...]
    v = jnp.clip(a, 0.0, 1.0) * (K - 1)
    k0 = jnp.minimum(jnp.floor(v), float(K - 2))
    frac = v - k0
    k0i = k0.astype(jnp.int32)
    i0_ref[...] = k0i * N + src_ref[...]
    w0_ref[...] = 1.0 - frac


def _prep(src2d, attr2d):
    sh = src2d.shape
    return pl.pallas_call(
        _prep_body,
        out_shape=(
            jax.ShapeDtypeStruct(sh, jnp.int32),
            jax.ShapeDtypeStruct(sh, jnp.float32),
        ),
    )(src2d, attr2d)


def _mm_first_body(x_ref, w_ref, out_ref):
    out_ref[0] = jnp.dot(x_ref[...], w_ref[0],
                         preferred_element_type=jnp.float32)


def _mm_first(x, wstack):
    kk, din, dout = wstack.shape
    grid = (N // _TN, kk)
    return pl.pallas_call(
        _mm_first_body,
        grid=grid,
        in_specs=[
            pl.BlockSpec((_TN, din), lambda n, k: (n, 0)),
            pl.BlockSpec((1, din, dout), lambda n, k: (k, 0, 0)),
        ],
        out_specs=pl.BlockSpec((1, _TN, dout), lambda n, k: (k, n, 0)),
        out_shape=jax.ShapeDtypeStruct((kk, N, dout), jnp.float32),
    )(x, wstack)


def _elu(v):
    return jnp.where(v > 0, v, jnp.exp(v) - 1.0)


def _mm_fused_body(agg_ref, r_ref, b_ref, w_ref, out_ref, h_ref):
    @pl.when(pl.program_id(1) == 0)
    def _():
        h_ref[...] = _elu(agg_ref[0] + agg_ref[1] + r_ref[...] + b_ref[...])

    out_ref[0] = jnp.dot(h_ref[...], w_ref[0],
                         preferred_element_type=jnp.float32)


def _mm_fused(agg, r, bias, wstack):
    kk, din, dout = wstack.shape
    grid = (N // _TN, kk)
    return pl.pallas_call(
        _mm_fused_body,
        grid=grid,
        in_specs=[
            pl.BlockSpec((2, _TN, din), lambda n, k: (0, n, 0)),
            pl.BlockSpec((_TN, din), lambda n, k: (n, 0)),
            pl.BlockSpec((1, din), lambda n, k: (0, 0)),
            pl.BlockSpec((1, din, dout), lambda n, k: (k, 0, 0)),
        ],
        out_specs=pl.BlockSpec((1, _TN, dout), lambda n, k: (k, n, 0)),
        out_shape=jax.ShapeDtypeStruct((kk, N, dout), jnp.float32),
        scratch_shapes=[pltpu.VMEM((_TN, din), jnp.float32)],
    )(agg, r, bias.reshape(1, din), wstack)


def _head_body(agg_ref, r_ref, b_ref, w1_ref, b1_ref, w2_ref, b2_ref, out_ref):
    h = _elu(agg_ref[0] + agg_ref[1] + r_ref[...] + b_ref[...])
    t = _elu(jnp.dot(h, w1_ref[...], preferred_element_type=jnp.float32)
             + b1_ref[...])
    logits = (jnp.dot(t, w2_ref[...], preferred_element_type=jnp.float32)
              + b2_ref[...])
    m = jnp.max(logits, axis=1, keepdims=True)
    lse = jnp.log(jnp.sum(jnp.exp(logits - m), axis=1, keepdims=True)) + m
    out_ref[...] = logits - lse


def _head(agg, r, bias, lin1_W, lin1_b, lin2_W, lin2_b):
    din = r.shape[1]
    d1 = lin1_W.shape[1]
    d2 = lin2_W.shape[1]
    grid = (N // _TN,)
    return pl.pallas_call(
        _head_body,
        grid=grid,
        in_specs=[
            pl.BlockSpec((2, _TN, din), lambda n: (0, n, 0)),
            pl.BlockSpec((_TN, din), lambda n: (n, 0)),
            pl.BlockSpec((1, din), lambda n: (0, 0)),
            pl.BlockSpec((din, d1), lambda n: (0, 0)),
            pl.BlockSpec((1, d1), lambda n: (0, 0)),
            pl.BlockSpec((d1, d2), lambda n: (0, 0)),
            pl.BlockSpec((1, d2), lambda n: (0, 0)),
        ],
        out_specs=pl.BlockSpec((_TN, d2), lambda n: (n, 0)),
        out_shape=jax.ShapeDtypeStruct((N, d2), jnp.float32),
    )(agg, r, bias.reshape(1, din), lin1_W, lin1_b.reshape(1, d1),
      lin2_W, lin2_b.reshape(1, d2))


_NBUF = 4


def _sc_agg_body(dout, y_hbm, i0_hbm, dst_hbm, w0_hbm,
                 out_hbm, i0_v, i1_v, dst_v, w0_v, rows0, rows1,
                 agg_sp, *sems):
    gsem = sems[:_NBUF]
    ssem = sems[_NBUF:]
    cid = lax.axis_index("c")
    sid = lax.axis_index("s")
    wid = cid * _NS + sid

    pltpu.sync_copy(i0_hbm.at[wid], i0_v)
    pltpu.sync_copy(dst_hbm.at[wid], dst_v)
    pltpu.sync_copy(w0_hbm.at[wid], w0_v)

    nq = dout // 16
    zv = jnp.zeros((16,), jnp.float32)

    def zr(r, c):
        for q in range(nq):
            rows0[0, r, pl.ds(q * 16, 16)] = zv
        return c

    lax.fori_loop(0, _C, zr, 0)
    for t in range(_NPS // _C):
        pltpu.sync_copy(rows0.at[0],
                        agg_sp.at[pl.ds(sid * _NPS + t * _C, _C)])

    def ini(r, c):
        for p in range(_C // 16):
            sl = pl.ds(p * 16, 16)
            i1_v[r, sl] = i0_v[r, sl] + N
        return c

    lax.fori_loop(0, _NCH, ini, 0)
    plsc.subcore_barrier()

    def issue(ch, b):
        pltpu.async_copy(y_hbm.at[i0_v.at[ch]], rows0.at[b], gsem[b])
        pltpu.async_copy(y_hbm.at[i1_v.at[ch]], rows1.at[b], gsem[b])

    for b in range(_NBUF - 1):
        issue(b, b)

    def chunk(b, g5):
        ch = g5 * _NBUF + b
        pltpu.make_async_copy(y_hbm.at[i0_v.at[ch]], rows0.at[b],
                              gsem[b]).wait()
        pltpu.make_async_copy(y_hbm.at[i1_v.at[ch]], rows1.at[b],
                              gsem[b]).wait()

        bb = (b + _NBUF - 1) % _NBUF

        @pl.when(ch + _NBUF - 1 < _NCH)
        def _():
            @pl.when(ch >= 1)
            def _():
                pltpu.make_async_copy(rows0.at[bb],
                                      agg_sp.at[dst_v.at[ch - 1]],
                                      ssem[bb]).wait()

            issue(ch + _NBUF - 1, bb)

        def grp(g, c2):
            wv0 = w0_v[ch, pl.ds(g * 16, 16)]
            wv1 = 1.0 - wv0
            for j in range(16):
                rr = g * 16 + j
                a = wv0[j]
                bbw = wv1[j]
                for q in range(nq):
                    sl = pl.ds(q * 16, 16)
                    rows0[b, rr, sl] = (a * rows0[b, rr, sl]
                                        + bbw * rows1[b, rr, sl])
            return c2

        lax.fori_loop(0, _C // 16, grp, 0)
        pltpu.async_copy(rows0.at[b], agg_sp.at[dst_v.at[ch]], ssem[b],
                         add=True)

    def outer(g5, carry):
        for b in range(_NBUF):
            chunk(b, g5)
        return carry

    lax.fori_loop(0, _NCH // _NBUF, outer, 0)
    for t in range(_NCH - (_NCH // _NBUF) * _NBUF):
        chunk(t, jnp.int32(_NCH // _NBUF))
    for b in range(_NBUF):
        pltpu.make_async_copy(rows0.at[b], agg_sp.at[dst_v.at[0]],
                              ssem[b]).wait()
    plsc.subcore_barrier()

    pltpu.sync_copy(agg_sp.at[pl.ds(sid * _NPS, _NPS)],
                    out_hbm.at[cid, pl.ds(sid * _NPS, _NPS)])


@functools.partial(jax.jit, static_argnames=("dout",))
def _sc_agg(y, i0, dst, w0, *, dout):
    mesh = plsc.VectorSubcoreMesh(core_axis_name="c", subcore_axis_name="s")
    body = functools.partial(_sc_agg_body, dout)
    return pl.kernel(
        body,
        out_type=pltpu.HBM((_NC, _NPAD, dout), jnp.float32),
        mesh=mesh,
        scratch_types=[
            pltpu.VMEM((_NCH, _C), jnp.int32),
            pltpu.VMEM((_NCH, _C), jnp.int32),
            pltpu.VMEM((_NCH, _C), jnp.int32),
            pltpu.VMEM((_NCH, _C), jnp.float32),
            pltpu.VMEM((_NBUF, _C, dout), jnp.float32),
            pltpu.VMEM((_NBUF, _C, dout), jnp.float32),
            pltpu.VMEM_SHARED((_NPAD, dout), jnp.float32),
        ] + [pltpu.SemaphoreType.DMA] * (2 * _NBUF),
        compiler_params=pltpu.CompilerParams(use_tc_tiling_on_sc=False),
    )(y, i0, dst, w0)


def kernel(x, edge_index, edge_attr,
           conv1_W, conv1_root, conv1_bias,
           conv2_W, conv2_root, conv2_bias,
           conv3_W, conv3_root, conv3_bias,
           conv4_W, conv4_root, conv4_bias,
           conv5_W, conv5_root, conv5_bias,
           conv6_W, conv6_root, conv6_bias,
           lin1_W, lin1_b, lin2_W, lin2_b):
    src = edge_index[0]
    dst = edge_index[1]
    attr = edge_attr[:, 0]

    i0, w0 = _prep(src.reshape(2500, 128), attr.reshape(2500, 128))
    i0 = i0.reshape(_NW, _NCH, _C)
    w0 = w0.reshape(_NW, _NCH, _C)
    dst2 = dst.reshape(_NW, _NCH, _C)

    convs = [
        (conv1_W, conv1_root, conv1_bias),
        (conv2_W, conv2_root, conv2_bias),
        (conv3_W, conv3_root, conv3_bias),
        (conv4_W, conv4_root, conv4_bias),
        (conv5_W, conv5_root, conv5_bias),
        (conv6_W, conv6_root, conv6_bias),
    ]

    wstack = jnp.concatenate([convs[0][0], convs[0][1][None]], axis=0)
    y = _mm_first(x, wstack)
    dout = wstack.shape[2]
    agg = _sc_agg(y.reshape(-1, dout), i0, dst2, w0, dout=dout)
    r = y[K]
    prev_bias = convs[0][2]

    for W, root, bias in convs[1:]:
        wstack = jnp.concatenate([W, root[None]], axis=0)
        y = _mm_fused(agg, r, prev_bias, wstack)
        dout = wstack.shape[2]
        agg = _sc_agg(y.reshape(-1, dout), i0, dst2, w0, dout=dout)
        r = y[K]
        prev_bias = bias

    return _head(agg, r, prev_bias, lin1_W, lin1_b, lin2_W, lin2_b)
